# baseline (speedup 1.0000x reference)
import functools

import jax
import jax.numpy as jnp
from jax import lax
from jax.experimental import pallas as pl
from jax.experimental.pallas import tpu as pltpu
from jax.experimental.pallas import tpu_sc as plsc

B = 16384
Z_DIM = 64
SMALL_D = 8
NC = 2
NS = 16
NW = NC * NS
BPW = B // NW
CHUNK = 128
NCHUNK = BPW // CHUNK


@functools.partial(
    pl.kernel,
    mesh=plsc.VectorSubcoreMesh(core_axis_name="c", subcore_axis_name="s"),
    out_type=(
        jax.ShapeDtypeStruct((B, Z_DIM), jnp.float32),
        jax.ShapeDtypeStruct((B, SMALL_D), jnp.float32),
    ),
    scratch_types=[
        pltpu.VMEM((NCHUNK, CHUNK), jnp.int32),
        pltpu.VMEM((BPW, Z_DIM), jnp.float32),
        pltpu.VMEM((BPW, SMALL_D), jnp.float32),
        pltpu.SemaphoreType.DMA,
    ],
    compiler_params=pltpu.CompilerParams(use_tc_tiling_on_sc=False),
)
def _sc_gather(idx_hbm, emb_hbm, small_hbm,
               z_out, small_out,
               idx_v, z_v, small_v, sem):
    wid = lax.axis_index("s") * NC + lax.axis_index("c")
    pltpu.sync_copy(idx_hbm.at[wid], idx_v)
    copies = []
    for c in range(NCHUNK):
        sl = pl.ds(c * CHUNK, CHUNK)
        copies.append(pltpu.async_copy(emb_hbm.at[idx_v.at[c]], z_v.at[sl], sem))
        copies.append(pltpu.async_copy(small_hbm.at[idx_v.at[c]], small_v.at[sl], sem))
    for cp in copies:
        cp.wait()
    base = wid * BPW
    pltpu.sync_copy(z_v, z_out.at[pl.ds(base, BPW)])
    pltpu.sync_copy(small_v, small_out.at[pl.ds(base, BPW)])


def kernel(piano_model, emb_w, inharm_w, detune_w):
    idx = piano_model.astype(jnp.int32).reshape(NW, NCHUNK, CHUNK)
    n = emb_w.shape[0]
    small = jnp.concatenate(
        [inharm_w, detune_w,
         jnp.zeros((n, SMALL_D - 2), jnp.float32)], axis=1)
    z, small_rows = _sc_gather(idx, emb_w, small)
    return (z[:, None, :],
            small_rows[:, None, 0:1],
            small_rows[:, None, 1:2])


# speedup vs baseline: 1.6760x; 1.6760x over previous
import functools

import jax
import jax.numpy as jnp
from jax import lax
from jax.experimental import pallas as pl
from jax.experimental.pallas import tpu as pltpu
from jax.experimental.pallas import tpu_sc as plsc

B = 16384
Z_DIM = 64
NC = 2
NS = 16
NW = NC * NS
BPW = B // NW
CHUNK = 128
NCHUNK = BPW // CHUNK
L = 16
NVEC = BPW // L


@functools.partial(
    pl.kernel,
    mesh=plsc.VectorSubcoreMesh(core_axis_name="c", subcore_axis_name="s"),
    out_type=(
        jax.ShapeDtypeStruct((B, Z_DIM), jnp.float32),
        jax.ShapeDtypeStruct((B,), jnp.float32),
        jax.ShapeDtypeStruct((B,), jnp.float32),
    ),
    scratch_types=[
        pltpu.VMEM((NCHUNK, CHUNK), jnp.int32),
        pltpu.VMEM((NCHUNK, CHUNK), jnp.int32),
        pltpu.VMEM((BPW, Z_DIM), jnp.float32),
        pltpu.VMEM((BPW, 8), jnp.float32),
        pltpu.VMEM((BPW, 8), jnp.float32),
        pltpu.VMEM((BPW,), jnp.float32),
        pltpu.VMEM((BPW,), jnp.float32),
        pltpu.SemaphoreType.DMA,
    ],
    compiler_params=pltpu.CompilerParams(
        use_tc_tiling_on_sc=False, needs_layout_passes=False),
)
def _sc_gather(idx_hbm, emb_hbm, inh8_hbm, det8_hbm,
               z_out, inh_out, det_out,
               idx_v, row_v, z_v, inh_rows, det_rows, inh_val, det_val, sem):
    wid = lax.axis_index("s") * NC + lax.axis_index("c")
    pltpu.sync_copy(idx_hbm.at[wid], idx_v)
    for c in range(NCHUNK):
        for k in range(CHUNK // L):
            sl = pl.ds(k * L, L)
            row_v[c, sl] = lax.shift_right_logical(idx_v[c, sl], 3)
    copies = []
    for c in range(NCHUNK):
        sl = pl.ds(c * CHUNK, CHUNK)
        copies.append(pltpu.async_copy(emb_hbm.at[idx_v.at[c]], z_v.at[sl], sem))
        copies.append(pltpu.async_copy(inh8_hbm.at[row_v.at[c]], inh_rows.at[sl], sem))
        copies.append(pltpu.async_copy(det8_hbm.at[row_v.at[c]], det_rows.at[sl], sem))
    for cp in copies:
        cp.wait()
    lane = lax.iota(jnp.int32, L)
    seven = jnp.full((L,), 7, jnp.int32)
    for j in range(NVEC):
        c, off = (j * L) // CHUNK, (j * L) % CHUNK
        cols = lax.bitwise_and(idx_v[c, pl.ds(off, L)], seven)
        rows = lane + j * L
        inh_val[pl.ds(j * L, L)] = plsc.load_gather(inh_rows, [rows, cols])
        det_val[pl.ds(j * L, L)] = plsc.load_gather(det_rows, [rows, cols])
    base = wid * BPW
    pltpu.sync_copy(z_v, z_out.at[pl.ds(base, BPW)])
    pltpu.sync_copy(inh_val, inh_out.at[pl.ds(base, BPW)])
    pltpu.sync_copy(det_val, det_out.at[pl.ds(base, BPW)])


def kernel(piano_model, emb_w, inharm_w, detune_w):
    idx = piano_model.astype(jnp.int32).reshape(NW, NCHUNK, CHUNK)
    n = emb_w.shape[0]
    inh8 = inharm_w.reshape(n // 8, 8)
    det8 = detune_w.reshape(n // 8, 8)
    z, inh, det = _sc_gather(idx, emb_w, inh8, det8)
    return (z[:, None, :],
            inh.reshape(B, 1, 1),
            det.reshape(B, 1, 1))


# speedup vs baseline: 1.8085x; 1.0791x over previous
import functools

import jax
import jax.numpy as jnp
from jax import lax
from jax.experimental import pallas as pl
from jax.experimental.pallas import tpu as pltpu
from jax.experimental.pallas import tpu_sc as plsc

B = 16384
Z_DIM = 64
ZP = 128
NC = 2
NS = 16
NW = NC * NS
BPW = B // NW
CHUNK = 128
NCHUNK = BPW // CHUNK


@functools.partial(
    pl.kernel,
    mesh=plsc.VectorSubcoreMesh(core_axis_name="c", subcore_axis_name="s"),
    out_type=(
        jax.ShapeDtypeStruct((B, ZP), jnp.float32),
        jax.ShapeDtypeStruct((B,), jnp.float32),
        jax.ShapeDtypeStruct((B,), jnp.float32),
    ),
    scratch_types=[
        pltpu.VMEM((BPW,), jnp.int32),
        pltpu.VMEM((BPW, ZP), jnp.float32),
        pltpu.VMEM((BPW,), jnp.float32),
        pltpu.VMEM((BPW,), jnp.float32),
        pltpu.SemaphoreType.DMA,
    ],
    compiler_params=pltpu.CompilerParams(use_tc_tiling_on_sc=True),
)
def _sc_gather(idx_hbm, emb_hbm, inh_hbm, det_hbm,
               z_out, inh_out, det_out,
               idx_v, z_v, inh_v, det_v, sem):
    wid = lax.axis_index("s") * NC + lax.axis_index("c")
    base = wid * BPW
    pltpu.sync_copy(idx_hbm.at[pl.ds(base, BPW)], idx_v)
    copies = []
    for c in range(NCHUNK):
        sl = pl.ds(c * CHUNK, CHUNK)
        copies.append(pltpu.async_copy(emb_hbm.at[idx_v.at[sl]], z_v.at[sl], sem))
        copies.append(pltpu.async_copy(inh_hbm.at[idx_v.at[sl]], inh_v.at[sl], sem))
        copies.append(pltpu.async_copy(det_hbm.at[idx_v.at[sl]], det_v.at[sl], sem))
    for cp in copies:
        cp.wait()
    pltpu.sync_copy(z_v, z_out.at[pl.ds(base, BPW)])
    pltpu.sync_copy(inh_v, inh_out.at[pl.ds(base, BPW)])
    pltpu.sync_copy(det_v, det_out.at[pl.ds(base, BPW)])


def kernel(piano_model, emb_w, inharm_w, detune_w):
    idx = piano_model.astype(jnp.int32)
    emb128 = jnp.pad(emb_w, ((0, 0), (0, ZP - Z_DIM)))
    z128, inh, det = _sc_gather(idx, emb128,
                                inharm_w.reshape(-1), detune_w.reshape(-1))
    return (z128[:, None, :Z_DIM],
            inh.reshape(B, 1, 1),
            det.reshape(B, 1, 1))
